# baseline (speedup 1.0000x reference)
import functools

import jax
import jax.numpy as jnp
from jax import lax
from jax.experimental import pallas as pl
from jax.experimental.pallas import tpu as pltpu
from jax.experimental.pallas import tpu_sc as plsc

N = 10000
D_IN = 256
D_OUT = 256
DEG = 16
E = N * DEG
L = 16
NC = 2
NS = 16
NW = NC * NS
NPT = -(-N // NW)
NCH = D_IN // L

_mesh = plsc.VectorSubcoreMesh(core_axis_name="c", subcore_axis_name="s")


def _i16(v):
    return jnp.full((L,), v, dtype=jnp.int32)


def _rsqrt(s):
    i = plsc.bitcast(s, jnp.int32)
    i = jnp.int32(0x5F3759DF) - (i >> 1)
    y = plsc.bitcast(i, jnp.float32)
    for _ in range(3):
        y = y * (jnp.float32(1.5) - jnp.float32(0.5) * s * y * y)
    return y


NB = 4
T = -(-(NPT + 1) // NB)
NNODE = T * NB
IDXN = NNODE + NB


@functools.partial(
    pl.kernel,
    out_type=jax.ShapeDtypeStruct((N + NW, 4 * D_IN), jnp.float32),
    mesh=_mesh,
    compiler_params=pltpu.CompilerParams(needs_layout_passes=False),
    scratch_types=[
        pltpu.VMEM((3 * N,), jnp.float32),
        pltpu.VMEM((3 * N,), jnp.float32),
        pltpu.VMEM((IDXN * DEG,), jnp.int32),
        pltpu.VMEM((NB, DEG, D_IN), jnp.float32),
        pltpu.VMEM((NB * 4 * D_IN,), jnp.float32),
        pltpu.VMEM((L,), jnp.float32),
        [pltpu.SemaphoreType.DMA] * NB,
        [pltpu.SemaphoreType.DMA] * NB,
    ],
)
def _sc_edge_stage(feats_hbm, ipos_hbm, opos_hbm, ext_hbm, nidx_hbm, agg_hbm,
                   ipos_v, opos_v, idx_v, rows_v, acc_v, ext_v,
                   sems, semsO):
    wid = lax.axis_index("s") * NC + lax.axis_index("c")
    base = wid * NPT
    count = jnp.minimum(NPT, N - base)

    pltpu.sync_copy(ipos_hbm, ipos_v)
    pltpu.sync_copy(opos_hbm, opos_v)
    pltpu.sync_copy(ext_hbm, ext_v)
    pltpu.sync_copy(nidx_hbm.at[pl.ds(base * DEG, IDXN * DEG)], idx_v)

    ext = ext_v[...]
    c0 = _i16(0)
    c1 = _i16(1)
    c2 = _i16(2)

    def issue(m, b, sem):
        pltpu.async_copy(feats_hbm.at[idx_v.at[pl.ds(m * DEG, DEG)]],
                         rows_v.at[b], sem)

    def wait(m, b, sem):
        pltpu.make_async_copy(feats_hbm.at[idx_v.at[pl.ds(m * DEG, DEG)]],
                              rows_v.at[b], sem).wait()

    def polar(n, node):
        idx = idx_v[pl.ds(n * DEG, DEG)]
        idx3 = idx * 3
        nq = jnp.minimum(node, N - 1)
        node3 = jnp.full((L,), 3 * nq, dtype=jnp.int32)
        px = plsc.load_gather(ipos_v, [idx3 + c0])
        py = plsc.load_gather(ipos_v, [idx3 + c1])
        pz = plsc.load_gather(ipos_v, [idx3 + c2])
        qx = plsc.load_gather(opos_v, [node3 + c0])
        qy = plsc.load_gather(opos_v, [node3 + c1])
        qz = plsc.load_gather(opos_v, [node3 + c2])
        dx = px - qx
        dy = py - qy
        dz = pz - qz
        s = dx * dx + dy * dy + dz * dz
        rinv = _rsqrt(s)
        r = s * rinv
        return [r / ext,
                dx * rinv,
                dz * rinv,
                dy * rinv]

    def accumulate(w, b):
        ob = b * (4 * D_IN)
        for h in range(2):
            w0k = [jnp.full((L,), w[2 * h][k]) for k in range(DEG)]
            w1k = [jnp.full((L,), w[2 * h + 1][k]) for k in range(DEG)]
            for c in range(NCH):
                row = rows_v[b, 0, pl.ds(c * L, L)]
                a0 = w0k[0] * row
                a1 = w1k[0] * row
                for k in range(1, DEG):
                    row = rows_v[b, k, pl.ds(c * L, L)]
                    a0 = a0 + w0k[k] * row
                    a1 = a1 + w1k[k] * row
                acc_v[pl.ds(ob + (2 * h) * D_IN + c * L, L)] = a0
                acc_v[pl.ds(ob + (2 * h + 1) * D_IN + c * L, L)] = a1

    def out_slice(b):
        return acc_v.at[pl.ds(b * (4 * D_IN), 4 * D_IN)]

    def out_dst(n):
        return agg_hbm.at[jnp.where(n < count, base + n, N + wid)]

    for j in range(NB - 1):
        issue(j, j, sems[j])

    def body(t, _):
        n0 = NB * t
        for j in range(NB):
            nj = n0 + j
            b = j
            pf = (j + NB - 1) % NB
            issue(nj + NB - 1, pf, sems[pf])
            wait(nj, b, sems[b])
            w = polar(nj, base + nj)

            @pl.when(t > 0)
            def _():
                pltpu.make_async_copy(out_slice(b), out_dst(nj), semsO[b]).wait()
            accumulate(w, b)
            pltpu.async_copy(out_slice(b), out_dst(nj), semsO[b])
        return 0

    lax.fori_loop(0, T, body, 0)
    for j in range(NB):
        pltpu.make_async_copy(out_slice(j), out_dst(j), semsO[j]).wait()
    for j in range(NB - 1):
        b = (NNODE + j) % NB
        pltpu.make_async_copy(feats_hbm.at[idx_v.at[pl.ds(0, DEG)]],
                              rows_v.at[b], sems[b]).wait()


def _mm_body(x_ref, w_ref, o_ref):
    o_ref[...] = jnp.dot(x_ref[...], w_ref[...],
                         precision=lax.Precision.HIGHEST,
                         preferred_element_type=jnp.float32)


def _matmul(agg, kflat):
    bm = 1000
    return pl.pallas_call(
        _mm_body,
        grid=(N // bm,),
        in_specs=[
            pl.BlockSpec((bm, 4 * D_IN), lambda i: (i, 0)),
            pl.BlockSpec((4 * D_IN, D_OUT), lambda i: (0, 0)),
        ],
        out_specs=pl.BlockSpec((bm, D_OUT), lambda i: (i, 0)),
        out_shape=jax.ShapeDtypeStruct((N, D_OUT), jnp.float32),
    )(agg, kflat)


def kernel(inp_features, inp_positions, out_positions, extents,
           neighbors_index, neighbors_row_splits, kernel):
    del neighbors_row_splits
    nidx = neighbors_index.astype(jnp.int32)
    nidx = jnp.pad(nidx, (0, (NW * NPT + IDXN) * DEG - E))
    ext16 = jnp.broadcast_to(extents.astype(jnp.float32), (L,))
    agg = _sc_edge_stage(inp_features, inp_positions.reshape(-1),
                         out_positions.reshape(-1), ext16, nidx)
    kflat = kernel.reshape(4 * D_IN, D_OUT)
    return _matmul(agg, kflat)


# speedup vs baseline: 1.0703x; 1.0703x over previous
import functools

import jax
import jax.numpy as jnp
from jax import lax
from jax.experimental import pallas as pl
from jax.experimental.pallas import tpu as pltpu
from jax.experimental.pallas import tpu_sc as plsc

N = 10000
D_IN = 256
D_OUT = 256
DEG = 16
E = N * DEG
L = 16
NC = 2
NS = 16
NW = NC * NS
NPT = -(-N // NW)
NCH = D_IN // L

_mesh = plsc.VectorSubcoreMesh(core_axis_name="c", subcore_axis_name="s")


def _i16(v):
    return jnp.full((L,), v, dtype=jnp.int32)


def _rsqrt(s):
    i = plsc.bitcast(s, jnp.int32)
    i = jnp.int32(0x5F3759DF) - (i >> 1)
    y = plsc.bitcast(i, jnp.float32)
    for _ in range(3):
        y = y * (jnp.float32(1.5) - jnp.float32(0.5) * s * y * y)
    return y


NB = 2
T = -(-(NPT + 1) // NB)
NNODE = T * NB
IDXN = NNODE + NB


@functools.partial(
    pl.kernel,
    out_type=jax.ShapeDtypeStruct((N + NW, 4 * D_IN), jnp.float32),
    mesh=_mesh,
    compiler_params=pltpu.CompilerParams(needs_layout_passes=False),
    scratch_types=[
        pltpu.VMEM((3 * N,), jnp.float32),
        pltpu.VMEM((3 * N,), jnp.float32),
        pltpu.VMEM((IDXN * DEG,), jnp.int32),
        pltpu.VMEM((NB, DEG, D_IN), jnp.float32),
        pltpu.VMEM((NB * 4 * D_IN,), jnp.float32),
        pltpu.VMEM((L,), jnp.float32),
        [pltpu.SemaphoreType.DMA] * NB,
        [pltpu.SemaphoreType.DMA] * NB,
    ],
)
def _sc_edge_stage(feats_hbm, ipos_hbm, opos_hbm, ext_hbm, nidx_hbm, agg_hbm,
                   ipos_v, opos_v, idx_v, rows_v, acc_v, ext_v,
                   sems, semsO):
    wid = lax.axis_index("s") * NC + lax.axis_index("c")
    base = wid * NPT
    count = jnp.minimum(NPT, N - base)

    pltpu.sync_copy(ipos_hbm, ipos_v)
    pltpu.sync_copy(opos_hbm, opos_v)
    pltpu.sync_copy(ext_hbm, ext_v)
    pltpu.sync_copy(nidx_hbm.at[pl.ds(base * DEG, IDXN * DEG)], idx_v)

    ext = ext_v[...]
    c0 = _i16(0)
    c1 = _i16(1)
    c2 = _i16(2)

    def issue(m, b, sem):
        pltpu.async_copy(feats_hbm.at[idx_v.at[pl.ds(m * DEG, DEG)]],
                         rows_v.at[b], sem)

    def wait(m, b, sem):
        pltpu.make_async_copy(feats_hbm.at[idx_v.at[pl.ds(m * DEG, DEG)]],
                              rows_v.at[b], sem).wait()

    def polar(n, node):
        idx = idx_v[pl.ds(n * DEG, DEG)]
        idx3 = idx * 3
        nq = jnp.minimum(node, N - 1)
        node3 = jnp.full((L,), 3 * nq, dtype=jnp.int32)
        px = plsc.load_gather(ipos_v, [idx3 + c0])
        py = plsc.load_gather(ipos_v, [idx3 + c1])
        pz = plsc.load_gather(ipos_v, [idx3 + c2])
        qx = plsc.load_gather(opos_v, [node3 + c0])
        qy = plsc.load_gather(opos_v, [node3 + c1])
        qz = plsc.load_gather(opos_v, [node3 + c2])
        dx = px - qx
        dy = py - qy
        dz = pz - qz
        s = dx * dx + dy * dy + dz * dz
        rinv = _rsqrt(s)
        r = s * rinv
        return [r / ext,
                dx * rinv,
                dz * rinv,
                dy * rinv]

    def accumulate(w, b):
        ob = b * (4 * D_IN)
        for h in range(2):
            w0k = [jnp.full((L,), w[2 * h][k]) for k in range(DEG)]
            w1k = [jnp.full((L,), w[2 * h + 1][k]) for k in range(DEG)]
            for c in range(NCH):
                row = rows_v[b, 0, pl.ds(c * L, L)]
                a0 = w0k[0] * row
                a1 = w1k[0] * row
                for k in range(1, DEG):
                    row = rows_v[b, k, pl.ds(c * L, L)]
                    a0 = a0 + w0k[k] * row
                    a1 = a1 + w1k[k] * row
                acc_v[pl.ds(ob + (2 * h) * D_IN + c * L, L)] = a0
                acc_v[pl.ds(ob + (2 * h + 1) * D_IN + c * L, L)] = a1

    def out_slice(b):
        return acc_v.at[pl.ds(b * (4 * D_IN), 4 * D_IN)]

    def out_dst(n):
        return agg_hbm.at[jnp.where(n < count, base + n, N + wid)]

    for j in range(NB - 1):
        issue(j, j, sems[j])

    def body(t, _):
        n0 = NB * t
        for j in range(NB):
            nj = n0 + j
            b = j
            pf = (j + NB - 1) % NB
            issue(nj + NB - 1, pf, sems[pf])
            wait(nj, b, sems[b])
            w = polar(nj, base + nj)

            @pl.when(t > 0)
            def _():
                pltpu.make_async_copy(out_slice(b), out_dst(nj), semsO[b]).wait()
            accumulate(w, b)
            pltpu.async_copy(out_slice(b), out_dst(nj), semsO[b])
        return 0

    lax.fori_loop(0, T, body, 0)
    for j in range(NB):
        pltpu.make_async_copy(out_slice(j), out_dst(j), semsO[j]).wait()
    for j in range(NB - 1):
        b = (NNODE + j) % NB
        pltpu.make_async_copy(feats_hbm.at[idx_v.at[pl.ds(0, DEG)]],
                              rows_v.at[b], sems[b]).wait()


def _mm_body(x_ref, w_ref, o_ref):
    o_ref[...] = jnp.dot(x_ref[...], w_ref[...],
                         precision=lax.Precision.HIGHEST,
                         preferred_element_type=jnp.float32)


def _matmul(agg, kflat):
    bm = 1000
    return pl.pallas_call(
        _mm_body,
        grid=(N // bm,),
        in_specs=[
            pl.BlockSpec((bm, 4 * D_IN), lambda i: (i, 0)),
            pl.BlockSpec((4 * D_IN, D_OUT), lambda i: (0, 0)),
        ],
        out_specs=pl.BlockSpec((bm, D_OUT), lambda i: (i, 0)),
        out_shape=jax.ShapeDtypeStruct((N, D_OUT), jnp.float32),
    )(agg, kflat)


def kernel(inp_features, inp_positions, out_positions, extents,
           neighbors_index, neighbors_row_splits, kernel):
    del neighbors_row_splits
    nidx = neighbors_index.astype(jnp.int32)
    nidx = jnp.pad(nidx, (0, (NW * NPT + IDXN) * DEG - E))
    ext16 = jnp.broadcast_to(extents.astype(jnp.float32), (L,))
    agg = _sc_edge_stage(inp_features, inp_positions.reshape(-1),
                         out_positions.reshape(-1), ext16, nidx)
    kflat = kernel.reshape(4 * D_IN, D_OUT)
    return _matmul(agg, kflat)
